# baseline (speedup 1.0000x reference)
import functools

import jax
import jax.numpy as jnp
from jax import lax
from jax.experimental import pallas as pl
from jax.experimental.pallas import tpu as pltpu
from jax.experimental.pallas import tpu_sc as plsc

_NC = 2
_NS = 16
_NW = _NC * _NS

_C = 1000
_U_ROWS = 1000000
_I_ROWS = 100000
_UG = _U_ROWS // _C
_IG = _I_ROWS // _C
_UJ = -(-_UG // _NW)
_IJ = -(-_IG // _NW)

_mesh = plsc.VectorSubcoreMesh(core_axis_name="c", subcore_axis_name="s")


@functools.partial(
    pl.kernel,
    out_type=[
        jax.ShapeDtypeStruct((_U_ROWS, 32), jnp.float32),
        jax.ShapeDtypeStruct((_I_ROWS, 32), jnp.float32),
    ],
    mesh=_mesh,
    scratch_types=[pltpu.VMEM((_C, 32), jnp.float32)],
    compiler_params=pltpu.CompilerParams(use_tc_tiling_on_sc=True),
)
def _sc_copy(u_in, i_in, u_out, i_out, buf):
    wid = lax.axis_index("s") * _NC + lax.axis_index("c")

    for j in range(_UJ):
        k = wid * _UJ + j

        @pl.when(k < _UG)
        def _():
            off = k * _C
            pltpu.sync_copy(u_in.at[pl.ds(off, _C)], buf)
            pltpu.sync_copy(buf, u_out.at[pl.ds(off, _C)])

    for j in range(_IJ):
        k = wid * _IJ + j

        @pl.when(k < _IG)
        def _():
            off = k * _C
            pltpu.sync_copy(i_in.at[pl.ds(off, _C)], buf)
            pltpu.sync_copy(buf, i_out.at[pl.ds(off, _C)])


def kernel(emb_user, emb_item):
    u, i = _sc_copy(emb_user, emb_item)
    return (u, i)


# speedup vs baseline: 12.0937x; 12.0937x over previous
import jax
import jax.numpy as jnp
from jax.experimental import pallas as pl
from jax.experimental.pallas import tpu as pltpu

_GRID = 16
_BU = 65536
_BI = 6656


def _copy_body(u_in, i_in, u_out, i_out):
    u_out[...] = u_in[...]
    i_out[...] = i_in[...]


def kernel(emb_user, emb_item):
    ut = emb_user.T
    it = emb_item.T
    u, i = pl.pallas_call(
        _copy_body,
        grid=(_GRID,),
        in_specs=[
            pl.BlockSpec((32, _BU), lambda g: (0, g)),
            pl.BlockSpec((32, _BI), lambda g: (0, g)),
        ],
        out_specs=[
            pl.BlockSpec((32, _BU), lambda g: (0, g)),
            pl.BlockSpec((32, _BI), lambda g: (0, g)),
        ],
        out_shape=[
            jax.ShapeDtypeStruct(ut.shape, ut.dtype),
            jax.ShapeDtypeStruct(it.shape, it.dtype),
        ],
    )(ut, it)
    return (u.T, i.T)
